# Initial kernel scaffold; baseline (speedup 1.0000x reference)
import functools

import jax
import jax.numpy as jnp
from jax import lax
from jax.experimental import pallas as pl
from jax.experimental.pallas import tpu as pltpu
from jax.experimental.pallas import tpu_sc as plsc

B = 4096
L = 200
D = 32
HALF = L // 2

NC, NS = 2, 16
NW = NC * NS
ROWS_PER_W = B // NW
HALVES_PER_W = ROWS_PER_W * 2

_mesh = plsc.VectorSubcoreMesh(core_axis_name="c", subcore_axis_name="s")


@functools.partial(
    pl.kernel,
    mesh=_mesh,
    out_type=jax.ShapeDtypeStruct((B, D), jnp.float32),
    scratch_types=[
        pltpu.VMEM((HALVES_PER_W, HALF), jnp.int32),
        pltpu.VMEM((2, HALF, D), jnp.float32),
        pltpu.VMEM((ROWS_PER_W, D), jnp.float32),
        pltpu.VMEM((D,), jnp.float32),
        pltpu.SemaphoreType.DMA,
        pltpu.SemaphoreType.DMA,
    ],
)
def _emb_sum(x_hbm, w_hbm, b_hbm, out_hbm, idx_v, buf_v, out_v, bias_v,
             sem0, sem1):
    wid = lax.axis_index("s") * NC + lax.axis_index("c")
    base = wid * HALVES_PER_W

    pltpu.sync_copy(x_hbm.at[pl.ds(base, HALVES_PER_W)], idx_v)
    pltpu.sync_copy(b_hbm, bias_v)
    bias0 = bias_v[pl.ds(0, 16)]
    bias1 = bias_v[pl.ds(16, 16)]

    def row_body(b, _):
        acc0 = bias0
        acc1 = bias1
        for h in range(2):
            pltpu.async_copy(
                w_hbm.at[idx_v.at[2 * b + h]], buf_v.at[h], sem0
            ).wait()

            def inner(j, accs):
                a0, a1 = accs
                a0 = a0 + buf_v[h, j, pl.ds(0, 16)]
                a1 = a1 + buf_v[h, j, pl.ds(16, 16)]
                return (a0, a1)

            acc0, acc1 = lax.fori_loop(0, HALF, inner, (acc0, acc1),
                                       unroll=4)
        out_v[b, pl.ds(0, 16)] = acc0
        out_v[b, pl.ds(16, 16)] = acc1
        return 0

    lax.fori_loop(0, ROWS_PER_W, row_body, 0)
    pltpu.sync_copy(out_v, out_hbm.at[pl.ds(wid * ROWS_PER_W, ROWS_PER_W)])


def kernel(x, emb_weight, emb_bias):
    x2 = x.reshape(B * 2, HALF)
    return _emb_sum(x2, emb_weight, emb_bias)


# speedup vs baseline: 1.9723x; 1.9723x over previous
import functools

import jax
import jax.numpy as jnp
from jax import lax
from jax.experimental import pallas as pl
from jax.experimental.pallas import tpu as pltpu
from jax.experimental.pallas import tpu_sc as plsc

B = 4096
L = 200
D = 32
HALF = L // 2

NC, NS = 2, 16
NW = NC * NS
ROWS_PER_W = B // NW
HALVES_PER_W = ROWS_PER_W * 2

_mesh = plsc.VectorSubcoreMesh(core_axis_name="c", subcore_axis_name="s")


@functools.partial(
    pl.kernel,
    mesh=_mesh,
    out_type=jax.ShapeDtypeStruct((B, D), jnp.float32),
    compiler_params=pltpu.CompilerParams(use_tc_tiling_on_sc=False),
    scratch_types=[
        pltpu.VMEM((HALVES_PER_W, HALF), jnp.int32),
        pltpu.VMEM((2, HALF, D), jnp.float32),
        pltpu.VMEM((ROWS_PER_W, D), jnp.float32),
        pltpu.VMEM((D,), jnp.float32),
        pltpu.SemaphoreType.DMA,
        pltpu.SemaphoreType.DMA,
    ],
)
def _emb_sum(x_hbm, w_hbm, b_hbm, out_hbm, idx_v, buf_v, out_v, bias_v,
             sem0, sem1):
    wid = lax.axis_index("s") * NC + lax.axis_index("c")
    base = wid * HALVES_PER_W

    pltpu.sync_copy(x_hbm.at[pl.ds(base, HALVES_PER_W)], idx_v)
    pltpu.sync_copy(b_hbm, bias_v)
    bias0 = bias_v[pl.ds(0, 16)]
    bias1 = bias_v[pl.ds(16, 16)]

    def row_body(b, _):
        acc0 = bias0
        acc1 = bias1
        for h in range(2):
            pltpu.async_copy(
                w_hbm.at[idx_v.at[2 * b + h]], buf_v.at[h], sem0
            ).wait()

            def inner(j, accs):
                a0, a1 = accs
                a0 = a0 + buf_v[h, j, pl.ds(0, 16)]
                a1 = a1 + buf_v[h, j, pl.ds(16, 16)]
                return (a0, a1)

            acc0, acc1 = lax.fori_loop(0, HALF, inner, (acc0, acc1),
                                       unroll=4)
        out_v[b, pl.ds(0, 16)] = acc0
        out_v[b, pl.ds(16, 16)] = acc1
        return 0

    lax.fori_loop(0, ROWS_PER_W, row_body, 0)
    pltpu.sync_copy(out_v, out_hbm.at[pl.ds(wid * ROWS_PER_W, ROWS_PER_W)])


def kernel(x, emb_weight, emb_bias):
    x2 = x.reshape(B * 2, HALF)
    return _emb_sum(x2, emb_weight, emb_bias)


# speedup vs baseline: 2.5104x; 1.2728x over previous
import functools

import jax
import jax.numpy as jnp
from jax import lax
from jax.experimental import pallas as pl
from jax.experimental.pallas import tpu as pltpu
from jax.experimental.pallas import tpu_sc as plsc

B = 4096
L = 200
D = 32
HALF = L // 2

NC, NS = 2, 16
NW = NC * NS
ROWS_PER_W = B // NW
HALVES_PER_W = ROWS_PER_W * 2
NBUF = 4

_mesh = plsc.VectorSubcoreMesh(core_axis_name="c", subcore_axis_name="s")


@functools.partial(
    pl.kernel,
    mesh=_mesh,
    out_type=jax.ShapeDtypeStruct((B, D), jnp.float32),
    compiler_params=pltpu.CompilerParams(use_tc_tiling_on_sc=False),
    scratch_types=[
        pltpu.VMEM((HALVES_PER_W, HALF), jnp.int32),
        pltpu.VMEM((NBUF, HALF, D), jnp.float32),
        pltpu.VMEM((ROWS_PER_W, D), jnp.float32),
        pltpu.VMEM((D,), jnp.float32),
    ] + [pltpu.SemaphoreType.DMA] * NBUF,
)
def _emb_sum(x_hbm, w_hbm, b_hbm, out_hbm, idx_v, buf_v, out_v, bias_v,
             *sems):
    wid = lax.axis_index("s") * NC + lax.axis_index("c")
    base = wid * HALVES_PER_W

    pltpu.sync_copy(x_hbm.at[pl.ds(base, HALVES_PER_W)], idx_v)
    pltpu.sync_copy(b_hbm, bias_v)
    bias0 = bias_v[pl.ds(0, 16)]
    bias1 = bias_v[pl.ds(16, 16)]

    def start(i, slot):
        pltpu.async_copy(w_hbm.at[idx_v.at[i]], buf_v.at[slot], sems[slot])

    def wait(slot):
        pltpu.make_async_copy(
            w_hbm.at[pl.ds(0, HALF)], buf_v.at[slot], sems[slot]
        ).wait()

    def accum(slot, accs):
        def inner(j, accs):
            a0, a1 = accs
            a0 = a0 + buf_v[slot, j, pl.ds(0, 16)]
            a1 = a1 + buf_v[slot, j, pl.ds(16, 16)]
            return (a0, a1)
        return lax.fori_loop(0, HALF, inner, accs, unroll=10)

    for slot in range(NBUF):
        start(slot, slot)

    @pl.loop(0, HALVES_PER_W, step=NBUF)
    def _(g):
        for half in range(NBUF // 2):
            accs = (bias0, bias1)
            for s2 in range(2):
                slot = half * 2 + s2
                wait(slot)
                accs = accum(slot, accs)

                @pl.when(g < HALVES_PER_W - NBUF)
                def _():
                    start(g + slot + NBUF, slot)

            row = g // 2 + half
            out_v[row, pl.ds(0, 16)] = accs[0]
            out_v[row, pl.ds(16, 16)] = accs[1]

    pltpu.sync_copy(out_v, out_hbm.at[pl.ds(wid * ROWS_PER_W, ROWS_PER_W)])


def kernel(x, emb_weight, emb_bias):
    x2 = x.reshape(B * 2, HALF)
    return _emb_sum(x2, emb_weight, emb_bias)
